# Initial kernel scaffold; baseline (speedup 1.0000x reference)
import functools

import jax
import jax.numpy as jnp
from jax import lax
from jax.experimental import pallas as pl
from jax.experimental.pallas import tpu as pltpu
from jax.experimental.pallas import tpu_sc as plsc

N = 10000
E = 320000
D = 128
H = 64

B_N = 500
B_E = 2000

NC = 2
NS = 16
EPT_B = E // (NC * NS)
KB = 80
EPT_D = E // NS
KD = 400
RPT = N // NS

_f32 = jnp.float32


def _stage_a_body(x_ref, pos_ref, win_ref, bin_ref, wlin_ref, wsrc_ref,
                  wdst_ref, wa1_ref, drec_ref, srec_ref):
    x = x_ref[...]
    h = jnp.maximum(jnp.dot(x, win_ref[...].T, preferred_element_type=_f32)
                    + bin_ref[...], 0.0)
    wda = jnp.dot(wa1_ref[...], wdst_ref[...], preferred_element_type=_f32)
    wsa = jnp.dot(wa1_ref[...], wsrc_ref[...], preferred_element_type=_f32)
    p = jnp.dot(h, wda.T, preferred_element_type=_f32)
    q = jnp.dot(h, wsa.T, preferred_element_type=_f32)
    v = jnp.dot(h, wlin_ref[...].T, preferred_element_type=_f32)
    pospad = jnp.concatenate(
        [pos_ref[...], jnp.zeros((B_N, H - 3), _f32)], axis=1)
    z128 = jnp.zeros((B_N, D), _f32)
    drec_ref[...] = jnp.concatenate([p, pospad, z128], axis=1)
    srec_ref[...] = jnp.concatenate([-q, -pospad, v], axis=1)


def _stage_a(x, pos, W_in, b_in2, W_lin, W_src, W_dst, Wa1):
    nblk = N // B_N
    full = pl.BlockSpec((D, D), lambda i: (0, 0))
    fullH = pl.BlockSpec((H, D), lambda i: (0, 0))
    bias = pl.BlockSpec((1, D), lambda i: (0, 0))
    return pl.pallas_call(
        _stage_a_body,
        grid=(nblk,),
        in_specs=[
            pl.BlockSpec((B_N, D), lambda i: (i, 0)),
            pl.BlockSpec((B_N, 3), lambda i: (i, 0)),
            full, bias, full, full, full, fullH,
        ],
        out_specs=[
            pl.BlockSpec((B_N, 2 * D), lambda i: (i, 0)),
            pl.BlockSpec((B_N, 2 * D), lambda i: (i, 0)),
        ],
        out_shape=[
            jax.ShapeDtypeStruct((N, 2 * D), _f32),
            jax.ShapeDtypeStruct((N, 2 * D), _f32),
        ],
    )(x, pos, W_in, b_in2, W_lin, W_src, W_dst, Wa1)


def _gather_body(drec, srec, src_h, dst_h, out, idxs, idxd, bufd, bufs,
                 sem1, sem2):
    c = lax.axis_index("c")
    s = lax.axis_index("s")
    wid = s * NC + c
    base = wid * EPT_B

    def chunk(i, carry):
        e0 = base + i * KB
        pltpu.sync_copy(dst_h.at[pl.ds(e0, KB)], idxd)
        pltpu.sync_copy(src_h.at[pl.ds(e0, KB)], idxs)
        cp1 = pltpu.async_copy(drec.at[idxd], bufd, sem1)
        cp2 = pltpu.async_copy(srec.at[idxs], bufs, sem2)
        cp1.wait()
        cp2.wait()

        def addrow(r, carry2):
            for cc in range(2 * D // 16):
                sl = pl.ds(cc * 16, 16)
                bufd[r, sl] = bufd[r, sl] + bufs[r, sl]
            return carry2

        lax.fori_loop(0, KB, addrow, 0)
        pltpu.sync_copy(bufd, out.at[pl.ds(e0, KB)])
        return carry

    lax.fori_loop(0, EPT_B // KB, chunk, 0)


_gather_kernel = functools.partial(
    pl.kernel,
    mesh=plsc.VectorSubcoreMesh(core_axis_name="c", subcore_axis_name="s"),
    out_type=jax.ShapeDtypeStruct((E, 2 * D), _f32),
    scratch_types=[
        pltpu.VMEM((KB,), jnp.int32),
        pltpu.VMEM((KB,), jnp.int32),
        pltpu.VMEM((KB, 2 * D), _f32),
        pltpu.VMEM((KB, 2 * D), _f32),
        pltpu.SemaphoreType.DMA,
        pltpu.SemaphoreType.DMA,
    ],
)(_gather_body)


def _stage_c_body(g_ref, wp1p_ref, bp1_ref, wp2_ref, bp2_ref, wa1_ref,
                  ba1_ref, wa2_ref, ba2_ref, ex_ref, exm_ref):
    g = g_ref[...]
    u0 = g[:, 0:H]
    dp = g[:, H:2 * H]
    vv = g[:, D:2 * D]
    t = jnp.maximum(jnp.dot(dp, wp1p_ref[...].T, preferred_element_type=_f32)
                    + bp1_ref[...], 0.0)
    delta = jnp.maximum(jnp.dot(t, wp2_ref[...].T, preferred_element_type=_f32)
                        + bp2_ref[...], 0.0)
    u = jnp.maximum(u0 + jnp.dot(delta, wa1_ref[...].T,
                                 preferred_element_type=_f32)
                    + ba1_ref[...], 0.0)
    alpha = jnp.maximum(jnp.dot(u, wa2_ref[...].T, preferred_element_type=_f32)
                        + ba2_ref[...], 0.0)
    ex = jnp.exp(alpha)
    ex_ref[...] = ex
    exm_ref[...] = ex * (vv + delta)


def _stage_c(g, Wp1p, bp1_2, Wp2, bp2_2, Wa1, ba1_2, Wa2, ba2_2):
    nblk = E // B_E
    wHH = pl.BlockSpec((H, H), lambda i: (0, 0))
    wDH = pl.BlockSpec((D, H), lambda i: (0, 0))
    wHD = pl.BlockSpec((H, D), lambda i: (0, 0))
    bH = pl.BlockSpec((1, H), lambda i: (0, 0))
    bD = pl.BlockSpec((1, D), lambda i: (0, 0))
    return pl.pallas_call(
        _stage_c_body,
        grid=(nblk,),
        in_specs=[
            pl.BlockSpec((B_E, 2 * D), lambda i: (i, 0)),
            wHH, bH, wDH, bD, wHD, bH, wDH, bD,
        ],
        out_specs=[
            pl.BlockSpec((B_E, D), lambda i: (i, 0)),
            pl.BlockSpec((B_E, D), lambda i: (i, 0)),
        ],
        out_shape=[
            jax.ShapeDtypeStruct((E, D), _f32),
            jax.ShapeDtypeStruct((E, D), _f32),
        ],
    )(g, Wp1p, bp1_2, Wp2, bp2_2, Wa1, ba1_2, Wa2, ba2_2)


def _scatter_body(ex_h, exm_h, dst_h, num_out, den_out, idxd, exb, exmb, zb,
                  numacc, denacc):
    c = lax.axis_index("c")
    s = lax.axis_index("s")
    col0 = c * H
    r0 = s * RPT

    zero16 = jnp.zeros((16,), _f32)

    def zrow(r, carry):
        for cc in range(H // 16):
            zb[r, pl.ds(cc * 16, 16)] = zero16
        return carry

    lax.fori_loop(0, RPT, zrow, 0)
    pltpu.sync_copy(zb, numacc.at[pl.ds(r0, RPT)])
    pltpu.sync_copy(zb, denacc.at[pl.ds(r0, RPT)])
    plsc.subcore_barrier()

    def chunk(i, carry):
        e0 = s * EPT_D + i * KD
        pltpu.sync_copy(dst_h.at[pl.ds(e0, KD)], idxd)
        pltpu.sync_copy(ex_h.at[pl.ds(e0, KD), pl.ds(col0, H)], exb)
        pltpu.sync_copy(exm_h.at[pl.ds(e0, KD), pl.ds(col0, H)], exmb)
        pltpu.sync_copy(exb, denacc.at[idxd], add=True)
        pltpu.sync_copy(exmb, numacc.at[idxd], add=True)
        return carry

    lax.fori_loop(0, EPT_D // KD, chunk, 0)
    plsc.subcore_barrier()
    pltpu.sync_copy(numacc.at[pl.ds(r0, RPT)], num_out.at[c, pl.ds(r0, RPT)])
    pltpu.sync_copy(denacc.at[pl.ds(r0, RPT)], den_out.at[c, pl.ds(r0, RPT)])


_scatter_kernel = functools.partial(
    pl.kernel,
    mesh=plsc.VectorSubcoreMesh(core_axis_name="c", subcore_axis_name="s"),
    out_type=(jax.ShapeDtypeStruct((NC, N, H), _f32),
              jax.ShapeDtypeStruct((NC, N, H), _f32)),
    scratch_types=[
        pltpu.VMEM((KD,), jnp.int32),
        pltpu.VMEM((KD, H), _f32),
        pltpu.VMEM((KD, H), _f32),
        pltpu.VMEM((RPT, H), _f32),
        pltpu.VMEM_SHARED((N, H), _f32),
        pltpu.VMEM_SHARED((N, H), _f32),
    ],
)(_scatter_body)


def _stage_e_body(nl_ref, nr_ref, dl_ref, dr_ref, wout_ref, bout_ref, o_ref):
    rl = nl_ref[0] / (dl_ref[0] + 1e-16)
    rr = nr_ref[0] / (dr_ref[0] + 1e-16)
    w = wout_ref[...]
    o = (jnp.dot(rl, w[:, 0:H].T, preferred_element_type=_f32)
         + jnp.dot(rr, w[:, H:D].T, preferred_element_type=_f32)
         + bout_ref[...])
    o_ref[...] = jnp.maximum(o, 0.0)


def _stage_e(num2, den2, W_out, b_out2):
    nblk = N // B_N
    left = pl.BlockSpec((1, B_N, H), lambda i: (0, i, 0))
    right = pl.BlockSpec((1, B_N, H), lambda i: (1, i, 0))
    return pl.pallas_call(
        _stage_e_body,
        grid=(nblk,),
        in_specs=[
            left, right, left, right,
            pl.BlockSpec((D, D), lambda i: (0, 0)),
            pl.BlockSpec((1, D), lambda i: (0, 0)),
        ],
        out_specs=pl.BlockSpec((B_N, D), lambda i: (i, 0)),
        out_shape=jax.ShapeDtypeStruct((N, D), _f32),
    )(num2, num2, den2, den2, W_out, b_out2)


def kernel(x, pos, edge_index, W_in, b_in, W_lin, W_src, W_dst, Wp1, bp1,
           Wp2, bp2, Wa1, ba1, Wa2, ba2, W_out, b_out):
    src = edge_index[0]
    dst = edge_index[1]
    Wp1p = jnp.concatenate([Wp1, jnp.zeros((H, H - 3), _f32)], axis=1)
    b_in2 = b_in.reshape(1, D)
    bp1_2 = bp1.reshape(1, H)
    bp2_2 = bp2.reshape(1, D)
    ba1_2 = ba1.reshape(1, H)
    ba2_2 = ba2.reshape(1, D)
    b_out2 = b_out.reshape(1, D)

    drec, srec = _stage_a(x, pos, W_in, b_in2, W_lin, W_src, W_dst, Wa1)
    g = _gather_kernel(drec, srec, src, dst)
    ex, exm = _stage_c(g, Wp1p, bp1_2, Wp2, bp2_2, Wa1, ba1_2, Wa2, ba2_2)
    num2, den2 = _scatter_kernel(ex, exm, dst)
    return _stage_e(num2, den2, W_out, b_out2)


# speedup vs baseline: 5.3478x; 5.3478x over previous
import functools

import jax
import jax.numpy as jnp
from jax import lax
from jax.experimental import pallas as pl
from jax.experimental.pallas import tpu as pltpu
from jax.experimental.pallas import tpu_sc as plsc

N = 10000
E = 320000
D = 128
H = 64

B_N = 1000
B_E = 2000

NC = 2
NS = 16
EPT_B = E // (NC * NS)
KB = 80
EPT_D = E // NS
KD = 200
RPT = 632
RPT_LAST = N - 15 * RPT

_f32 = jnp.float32


def _stage_a_body(x_ref, pos_ref, win_ref, bin_ref, wlin_ref, wsrc_ref,
                  wdst_ref, wa1_ref, drec_ref, srec_ref):
    x = x_ref[...]
    h = jnp.maximum(jnp.dot(x, win_ref[...].T, preferred_element_type=_f32)
                    + bin_ref[...], 0.0)
    wda = jnp.dot(wa1_ref[...], wdst_ref[...], preferred_element_type=_f32)
    wsa = jnp.dot(wa1_ref[...], wsrc_ref[...], preferred_element_type=_f32)
    p = jnp.dot(h, wda.T, preferred_element_type=_f32)
    q = jnp.dot(h, wsa.T, preferred_element_type=_f32)
    v = jnp.dot(h, wlin_ref[...].T, preferred_element_type=_f32)
    pospad = jnp.concatenate(
        [pos_ref[...], jnp.zeros((B_N, H - 3), _f32)], axis=1)
    z128 = jnp.zeros((B_N, D), _f32)
    drec_ref[...] = jnp.concatenate([p, pospad, z128], axis=1)
    srec_ref[...] = jnp.concatenate([-q, -pospad, v], axis=1)


def _stage_a(x, pos, W_in, b_in2, W_lin, W_src, W_dst, Wa1):
    nblk = N // B_N
    full = pl.BlockSpec((D, D), lambda i: (0, 0))
    fullH = pl.BlockSpec((H, D), lambda i: (0, 0))
    bias = pl.BlockSpec((1, D), lambda i: (0, 0))
    return pl.pallas_call(
        _stage_a_body,
        grid=(nblk,),
        in_specs=[
            pl.BlockSpec((B_N, D), lambda i: (i, 0)),
            pl.BlockSpec((B_N, 3), lambda i: (i, 0)),
            full, bias, full, full, full, fullH,
        ],
        out_specs=[
            pl.BlockSpec((B_N, 2 * D), lambda i: (i, 0)),
            pl.BlockSpec((B_N, 2 * D), lambda i: (i, 0)),
        ],
        out_shape=[
            jax.ShapeDtypeStruct((N, 2 * D), _f32),
            jax.ShapeDtypeStruct((N, 2 * D), _f32),
        ],
    )(x, pos, W_in, b_in2, W_lin, W_src, W_dst, Wa1)


def _gather_body(drec, srec, src_h, dst_h, out, idxs, idxd, bufd, bufs,
                 sem1, sem2):
    c = lax.axis_index("c")
    s = lax.axis_index("s")
    wid = s * NC + c
    base = wid * EPT_B

    def chunk(i, carry):
        e0 = base + i * KB
        pltpu.sync_copy(dst_h.at[pl.ds(e0, KB)], idxd)
        pltpu.sync_copy(src_h.at[pl.ds(e0, KB)], idxs)
        cp1 = pltpu.async_copy(drec.at[idxd], bufd, sem1)
        cp2 = pltpu.async_copy(srec.at[idxs], bufs, sem2)
        cp1.wait()
        cp2.wait()

        def addrow(r, carry2):
            for cc in range(2 * D // 16):
                sl = pl.ds(cc * 16, 16)
                bufd[r, sl] = bufd[r, sl] + bufs[r, sl]
            return carry2

        lax.fori_loop(0, KB, addrow, 0)
        pltpu.sync_copy(bufd, out.at[pl.ds(e0, KB)])
        return carry

    lax.fori_loop(0, EPT_B // KB, chunk, 0)


_gather_kernel = functools.partial(
    pl.kernel,
    mesh=plsc.VectorSubcoreMesh(core_axis_name="c", subcore_axis_name="s"),
    out_type=jax.ShapeDtypeStruct((E, 2 * D), _f32),
    scratch_types=[
        pltpu.VMEM((KB,), jnp.int32),
        pltpu.VMEM((KB,), jnp.int32),
        pltpu.VMEM((KB, 2 * D), _f32),
        pltpu.VMEM((KB, 2 * D), _f32),
        pltpu.SemaphoreType.DMA,
        pltpu.SemaphoreType.DMA,
    ],
)(_gather_body)


def _stage_c_body(g_ref, wp1p_ref, bp1_ref, wp2_ref, bp2_ref, wa1_ref,
                  ba1_ref, wa2_ref, ba2_ref, cat_ref):
    g = g_ref[...]
    u0 = g[:, 0:H]
    dp = g[:, H:2 * H]
    vv = g[:, D:2 * D]
    t = jnp.maximum(jnp.dot(dp, wp1p_ref[...].T, preferred_element_type=_f32)
                    + bp1_ref[...], 0.0)
    delta = jnp.maximum(jnp.dot(t, wp2_ref[...].T, preferred_element_type=_f32)
                        + bp2_ref[...], 0.0)
    u = jnp.maximum(u0 + jnp.dot(delta, wa1_ref[...].T,
                                 preferred_element_type=_f32)
                    + ba1_ref[...], 0.0)
    alpha = jnp.maximum(jnp.dot(u, wa2_ref[...].T, preferred_element_type=_f32)
                        + ba2_ref[...], 0.0)
    ex = jnp.exp(alpha)
    exm = ex * (vv + delta)
    cat_ref[0] = jnp.concatenate([exm[:, 0:H], ex[:, 0:H]], axis=1)
    cat_ref[1] = jnp.concatenate([exm[:, H:D], ex[:, H:D]], axis=1)


def _stage_c(g, Wp1p, bp1_2, Wp2, bp2_2, Wa1, ba1_2, Wa2, ba2_2):
    nblk = E // B_E
    wHH = pl.BlockSpec((H, H), lambda i: (0, 0))
    wDH = pl.BlockSpec((D, H), lambda i: (0, 0))
    wHD = pl.BlockSpec((H, D), lambda i: (0, 0))
    bH = pl.BlockSpec((1, H), lambda i: (0, 0))
    bD = pl.BlockSpec((1, D), lambda i: (0, 0))
    return pl.pallas_call(
        _stage_c_body,
        grid=(nblk,),
        in_specs=[
            pl.BlockSpec((B_E, 2 * D), lambda i: (i, 0)),
            wHH, bH, wDH, bD, wHD, bH, wDH, bD,
        ],
        out_specs=pl.BlockSpec((2, B_E, D), lambda i: (0, i, 0)),
        out_shape=jax.ShapeDtypeStruct((2, E, D), _f32),
    )(g, Wp1p, bp1_2, Wp2, bp2_2, Wa1, ba1_2, Wa2, ba2_2)


def _scatter_body(cat_h, dst_h, zeros_h, acc_out, idxd, catb, acc):
    c = lax.axis_index("c")
    s = lax.axis_index("s")
    r0 = s * RPT

    @pl.when(s < NS - 1)
    def _():
        pltpu.sync_copy(zeros_h.at[pl.ds(r0, RPT)], acc.at[pl.ds(r0, RPT)])

    @pl.when(s == NS - 1)
    def _():
        pltpu.sync_copy(zeros_h.at[pl.ds(r0, RPT_LAST)],
                        acc.at[pl.ds(r0, RPT_LAST)])

    plsc.subcore_barrier()

    def chunk(i, carry):
        e0 = s * EPT_D + i * KD
        pltpu.sync_copy(dst_h.at[pl.ds(e0, KD)], idxd)
        pltpu.sync_copy(cat_h.at[c, pl.ds(e0, KD)], catb)
        pltpu.sync_copy(catb, acc.at[idxd], add=True)
        return carry

    lax.fori_loop(0, EPT_D // KD, chunk, 0)
    plsc.subcore_barrier()

    @pl.when(s < NS - 1)
    def _():
        pltpu.sync_copy(acc.at[pl.ds(r0, RPT)], acc_out.at[c, pl.ds(r0, RPT)])

    @pl.when(s == NS - 1)
    def _():
        pltpu.sync_copy(acc.at[pl.ds(r0, RPT_LAST)],
                        acc_out.at[c, pl.ds(r0, RPT_LAST)])


_scatter_kernel = functools.partial(
    pl.kernel,
    mesh=plsc.VectorSubcoreMesh(core_axis_name="c", subcore_axis_name="s"),
    out_type=jax.ShapeDtypeStruct((NC, N, D), _f32),
    scratch_types=[
        pltpu.VMEM((KD,), jnp.int32),
        pltpu.VMEM((KD, D), _f32),
        pltpu.VMEM_SHARED((N, D), _f32),
    ],
)(_scatter_body)


def _stage_e_body(al_ref, ar_ref, wout_ref, bout_ref, o_ref):
    al = al_ref[0]
    ar = ar_ref[0]
    rl = al[:, 0:H] / (al[:, H:D] + 1e-16)
    rr = ar[:, 0:H] / (ar[:, H:D] + 1e-16)
    w = wout_ref[...]
    o = (jnp.dot(rl, w[:, 0:H].T, preferred_element_type=_f32)
         + jnp.dot(rr, w[:, H:D].T, preferred_element_type=_f32)
         + bout_ref[...])
    o_ref[...] = jnp.maximum(o, 0.0)


def _stage_e(acc2, W_out, b_out2):
    nblk = N // B_N
    left = pl.BlockSpec((1, B_N, D), lambda i: (0, i, 0))
    right = pl.BlockSpec((1, B_N, D), lambda i: (1, i, 0))
    return pl.pallas_call(
        _stage_e_body,
        grid=(nblk,),
        in_specs=[
            left, right,
            pl.BlockSpec((D, D), lambda i: (0, 0)),
            pl.BlockSpec((1, D), lambda i: (0, 0)),
        ],
        out_specs=pl.BlockSpec((B_N, D), lambda i: (i, 0)),
        out_shape=jax.ShapeDtypeStruct((N, D), _f32),
    )(acc2, acc2, W_out, b_out2)


def kernel(x, pos, edge_index, W_in, b_in, W_lin, W_src, W_dst, Wp1, bp1,
           Wp2, bp2, Wa1, ba1, Wa2, ba2, W_out, b_out):
    src = edge_index[0]
    dst = edge_index[1]
    Wp1p = jnp.concatenate([Wp1, jnp.zeros((H, H - 3), _f32)], axis=1)
    b_in2 = b_in.reshape(1, D)
    bp1_2 = bp1.reshape(1, H)
    bp2_2 = bp2.reshape(1, D)
    ba1_2 = ba1.reshape(1, H)
    ba2_2 = ba2.reshape(1, D)
    b_out2 = b_out.reshape(1, D)

    drec, srec = _stage_a(x, pos, W_in, b_in2, W_lin, W_src, W_dst, Wa1)
    g = _gather_kernel(drec, srec, src, dst)
    cat = _stage_c(g, Wp1p, bp1_2, Wp2, bp2_2, Wa1, ba1_2, Wa2, ba2_2)
    zeros_n = jnp.zeros((N, D), _f32)
    acc2 = _scatter_kernel(cat, dst, zeros_n)
    return _stage_e(acc2, W_out, b_out2)


# speedup vs baseline: 6.1173x; 1.1439x over previous
import functools

import jax
import jax.numpy as jnp
from jax import lax
from jax.experimental import pallas as pl
from jax.experimental.pallas import tpu as pltpu
from jax.experimental.pallas import tpu_sc as plsc

N = 10000
E = 320000
D = 128
H = 64

B_N = 1000
B_E = 2000

NC = 2
NS = 16
EPT_B = E // (NC * NS)
KB = 80
EPT_D = E // NS
KD = 80
RPT = 632
RPT_LAST = N - 15 * RPT

_f32 = jnp.float32


def _stage_a_body(x_ref, pos_ref, win_ref, bin_ref, wlin_ref, wsrc_ref,
                  wdst_ref, wa1_ref, drec_ref, srec_ref):
    x = x_ref[...]
    h = jnp.maximum(jnp.dot(x, win_ref[...].T, preferred_element_type=_f32)
                    + bin_ref[...], 0.0)
    wda = jnp.dot(wa1_ref[...], wdst_ref[...], preferred_element_type=_f32)
    wsa = jnp.dot(wa1_ref[...], wsrc_ref[...], preferred_element_type=_f32)
    p = jnp.dot(h, wda.T, preferred_element_type=_f32)
    q = jnp.dot(h, wsa.T, preferred_element_type=_f32)
    v = jnp.dot(h, wlin_ref[...].T, preferred_element_type=_f32)
    pospad = jnp.concatenate(
        [pos_ref[...], jnp.zeros((B_N, H - 3), _f32)], axis=1)
    drec_ref[...] = jnp.concatenate([p, pospad], axis=1)
    srec_ref[...] = jnp.concatenate([-q, -pospad, v], axis=1)


def _stage_a(x, pos, W_in, b_in2, W_lin, W_src, W_dst, Wa1):
    nblk = N // B_N
    full = pl.BlockSpec((D, D), lambda i: (0, 0))
    fullH = pl.BlockSpec((H, D), lambda i: (0, 0))
    bias = pl.BlockSpec((1, D), lambda i: (0, 0))
    return pl.pallas_call(
        _stage_a_body,
        grid=(nblk,),
        in_specs=[
            pl.BlockSpec((B_N, D), lambda i: (i, 0)),
            pl.BlockSpec((B_N, 3), lambda i: (i, 0)),
            full, bias, full, full, full, fullH,
        ],
        out_specs=[
            pl.BlockSpec((B_N, D), lambda i: (i, 0)),
            pl.BlockSpec((B_N, 2 * D), lambda i: (i, 0)),
        ],
        out_shape=[
            jax.ShapeDtypeStruct((N, D), _f32),
            jax.ShapeDtypeStruct((N, 2 * D), _f32),
        ],
    )(x, pos, W_in, b_in2, W_lin, W_src, W_dst, Wa1)


def _gather_body(drec, srec, src_h, dst_h, out,
                 idxs0, idxd0, bufd0, bufs0,
                 idxs1, idxd1, bufd1, bufs1,
                 semg0, semg1, semw0, semw1):
    c = lax.axis_index("c")
    s = lax.axis_index("s")
    wid = s * NC + c
    base = wid * EPT_B
    nchunks = EPT_B // KB

    slots = ((idxs0, idxd0, bufd0, bufs0, semg0, semw0),
             (idxs1, idxd1, bufd1, bufs1, semg1, semw1))

    def issue_gathers(slot, ci):
        idxs, idxd, bufd, bufs, semg, _ = slots[slot]
        e0 = base + ci * KB
        pltpu.sync_copy(dst_h.at[pl.ds(e0, KB)], idxd)
        pltpu.sync_copy(src_h.at[pl.ds(e0, KB)], idxs)
        pltpu.async_copy(drec.at[idxd], bufd, semg)
        pltpu.async_copy(srec.at[idxs], bufs, semg)

    def wait_gathers(slot):
        idxs, idxd, bufd, bufs, semg, _ = slots[slot]
        pltpu.make_async_copy(drec.at[idxd], bufd, semg).wait()
        pltpu.make_async_copy(srec.at[idxs], bufs, semg).wait()

    def add_rows(slot):
        _, _, bufd, bufs, _, _ = slots[slot]

        def addrow(r, carry2):
            for cc in range(D // 16):
                sl = pl.ds(cc * 16, 16)
                bufs[r, sl] = bufs[r, sl] + bufd[r, sl]
            return carry2

        lax.fori_loop(0, KB, addrow, 0)

    def issue_write(slot, ci):
        _, _, _, bufs, _, semw = slots[slot]
        e0 = base + ci * KB
        pltpu.async_copy(bufs, out.at[pl.ds(e0, KB)], semw)

    def wait_write(slot):
        _, _, _, bufs, _, semw = slots[slot]
        pltpu.make_async_copy(bufs, out.at[pl.ds(base, KB)], semw).wait()

    issue_gathers(0, 0)

    def pair(i, carry):
        c0 = 2 * i
        c1 = c0 + 1

        @pl.when(i > 0)
        def _():
            wait_write(1)

        issue_gathers(1, c1)
        wait_gathers(0)
        add_rows(0)
        issue_write(0, c0)
        wait_write(0)

        @pl.when(c0 + 2 < nchunks)
        def _():
            issue_gathers(0, c0 + 2)

        wait_gathers(1)
        add_rows(1)
        issue_write(1, c1)
        return carry

    lax.fori_loop(0, nchunks // 2, pair, 0)
    wait_write(1)
    wait_gathers(0)
    add_rows(0)
    pltpu.sync_copy(bufs0, out.at[pl.ds(base + (nchunks - 1) * KB, KB)])


_gather_kernel = functools.partial(
    pl.kernel,
    mesh=plsc.VectorSubcoreMesh(core_axis_name="c", subcore_axis_name="s"),
    out_type=jax.ShapeDtypeStruct((E, 2 * D), _f32),
    scratch_types=[
        pltpu.VMEM((KB,), jnp.int32),
        pltpu.VMEM((KB,), jnp.int32),
        pltpu.VMEM((KB, D), _f32),
        pltpu.VMEM((KB, 2 * D), _f32),
        pltpu.VMEM((KB,), jnp.int32),
        pltpu.VMEM((KB,), jnp.int32),
        pltpu.VMEM((KB, D), _f32),
        pltpu.VMEM((KB, 2 * D), _f32),
        pltpu.SemaphoreType.DMA,
        pltpu.SemaphoreType.DMA,
        pltpu.SemaphoreType.DMA,
        pltpu.SemaphoreType.DMA,
    ],
)(_gather_body)


def _stage_c_body(g_ref, wp1p_ref, bp1_ref, wp2_ref, bp2_ref, wa1_ref,
                  ba1_ref, wa2_ref, ba2_ref, cat_ref):
    g = g_ref[...]
    u0 = g[:, 0:H]
    dp = g[:, H:2 * H]
    vv = g[:, D:2 * D]
    t = jnp.maximum(jnp.dot(dp, wp1p_ref[...].T, preferred_element_type=_f32)
                    + bp1_ref[...], 0.0)
    delta = jnp.maximum(jnp.dot(t, wp2_ref[...].T, preferred_element_type=_f32)
                        + bp2_ref[...], 0.0)
    u = jnp.maximum(u0 + jnp.dot(delta, wa1_ref[...].T,
                                 preferred_element_type=_f32)
                    + ba1_ref[...], 0.0)
    alpha = jnp.maximum(jnp.dot(u, wa2_ref[...].T, preferred_element_type=_f32)
                        + ba2_ref[...], 0.0)
    ex = jnp.exp(alpha)
    exm = ex * (vv + delta)
    cat_ref[0] = jnp.concatenate([exm[:, 0:H], ex[:, 0:H]], axis=1)
    cat_ref[1] = jnp.concatenate([exm[:, H:D], ex[:, H:D]], axis=1)


def _stage_c(g, Wp1p, bp1_2, Wp2, bp2_2, Wa1, ba1_2, Wa2, ba2_2):
    nblk = E // B_E
    wHH = pl.BlockSpec((H, H), lambda i: (0, 0))
    wDH = pl.BlockSpec((D, H), lambda i: (0, 0))
    wHD = pl.BlockSpec((H, D), lambda i: (0, 0))
    bH = pl.BlockSpec((1, H), lambda i: (0, 0))
    bD = pl.BlockSpec((1, D), lambda i: (0, 0))
    return pl.pallas_call(
        _stage_c_body,
        grid=(nblk,),
        in_specs=[
            pl.BlockSpec((B_E, 2 * D), lambda i: (i, 0)),
            wHH, bH, wDH, bD, wHD, bH, wDH, bD,
        ],
        out_specs=pl.BlockSpec((2, B_E, D), lambda i: (0, i, 0)),
        out_shape=jax.ShapeDtypeStruct((2, E, D), _f32),
    )(g, Wp1p, bp1_2, Wp2, bp2_2, Wa1, ba1_2, Wa2, ba2_2)


def _scatter_body(cat_h, dst_h, zeros_h, acc_out, idxd0, catb0, idxd1, catb1,
                  acc, seml0, seml1):
    c = lax.axis_index("c")
    s = lax.axis_index("s")
    r0 = s * RPT

    @pl.when(s < NS - 1)
    def _():
        pltpu.sync_copy(zeros_h.at[pl.ds(r0, RPT)], acc.at[pl.ds(r0, RPT)])

    @pl.when(s == NS - 1)
    def _():
        pltpu.sync_copy(zeros_h.at[pl.ds(r0, RPT_LAST)],
                        acc.at[pl.ds(r0, RPT_LAST)])

    plsc.subcore_barrier()

    nchunks = EPT_D // KD
    slots = ((idxd0, catb0, seml0), (idxd1, catb1, seml1))

    def issue_load(slot, ci):
        idxd, catb, seml = slots[slot]
        e0 = s * EPT_D + ci * KD
        pltpu.sync_copy(dst_h.at[pl.ds(e0, KD)], idxd)
        pltpu.async_copy(cat_h.at[c, pl.ds(e0, KD)], catb, seml)

    def wait_load(slot):
        idxd, catb, seml = slots[slot]
        pltpu.make_async_copy(cat_h.at[c, pl.ds(0, KD)], catb, seml).wait()

    def scatter(slot):
        idxd, catb, _ = slots[slot]
        pltpu.sync_copy(catb, acc.at[idxd], add=True)

    issue_load(0, 0)

    def pair(i, carry):
        c0 = 2 * i
        issue_load(1, c0 + 1)
        wait_load(0)
        scatter(0)

        @pl.when(c0 + 2 < nchunks)
        def _():
            issue_load(0, c0 + 2)

        wait_load(1)
        scatter(1)
        return carry

    lax.fori_loop(0, nchunks // 2, pair, 0)
    plsc.subcore_barrier()

    @pl.when(s < NS - 1)
    def _():
        pltpu.sync_copy(acc.at[pl.ds(r0, RPT)], acc_out.at[c, pl.ds(r0, RPT)])

    @pl.when(s == NS - 1)
    def _():
        pltpu.sync_copy(acc.at[pl.ds(r0, RPT_LAST)],
                        acc_out.at[c, pl.ds(r0, RPT_LAST)])


_scatter_kernel = functools.partial(
    pl.kernel,
    mesh=plsc.VectorSubcoreMesh(core_axis_name="c", subcore_axis_name="s"),
    out_type=jax.ShapeDtypeStruct((NC, N, D), _f32),
    scratch_types=[
        pltpu.VMEM((KD,), jnp.int32),
        pltpu.VMEM((KD, D), _f32),
        pltpu.VMEM((KD,), jnp.int32),
        pltpu.VMEM((KD, D), _f32),
        pltpu.VMEM_SHARED((N, D), _f32),
        pltpu.SemaphoreType.DMA,
        pltpu.SemaphoreType.DMA,
    ],
)(_scatter_body)


def _stage_e_body(al_ref, ar_ref, wout_ref, bout_ref, o_ref):
    al = al_ref[0]
    ar = ar_ref[0]
    rl = al[:, 0:H] / (al[:, H:D] + 1e-16)
    rr = ar[:, 0:H] / (ar[:, H:D] + 1e-16)
    w = wout_ref[...]
    o = (jnp.dot(rl, w[:, 0:H].T, preferred_element_type=_f32)
         + jnp.dot(rr, w[:, H:D].T, preferred_element_type=_f32)
         + bout_ref[...])
    o_ref[...] = jnp.maximum(o, 0.0)


def _stage_e(acc2, W_out, b_out2):
    nblk = N // B_N
    left = pl.BlockSpec((1, B_N, D), lambda i: (0, i, 0))
    right = pl.BlockSpec((1, B_N, D), lambda i: (1, i, 0))
    return pl.pallas_call(
        _stage_e_body,
        grid=(nblk,),
        in_specs=[
            left, right,
            pl.BlockSpec((D, D), lambda i: (0, 0)),
            pl.BlockSpec((1, D), lambda i: (0, 0)),
        ],
        out_specs=pl.BlockSpec((B_N, D), lambda i: (i, 0)),
        out_shape=jax.ShapeDtypeStruct((N, D), _f32),
    )(acc2, acc2, W_out, b_out2)


def kernel(x, pos, edge_index, W_in, b_in, W_lin, W_src, W_dst, Wp1, bp1,
           Wp2, bp2, Wa1, ba1, Wa2, ba2, W_out, b_out):
    src = edge_index[0]
    dst = edge_index[1]
    Wp1p = jnp.concatenate([Wp1, jnp.zeros((H, H - 3), _f32)], axis=1)
    b_in2 = b_in.reshape(1, D)
    bp1_2 = bp1.reshape(1, H)
    bp2_2 = bp2.reshape(1, D)
    ba1_2 = ba1.reshape(1, H)
    ba2_2 = ba2.reshape(1, D)
    b_out2 = b_out.reshape(1, D)

    drec, srec = _stage_a(x, pos, W_in, b_in2, W_lin, W_src, W_dst, Wa1)
    g = _gather_kernel(drec, srec, src, dst)
    cat = _stage_c(g, Wp1p, bp1_2, Wp2, bp2_2, Wa1, ba1_2, Wa2, ba2_2)
    zeros_n = jnp.zeros((N, D), _f32)
    acc2 = _scatter_kernel(cat, dst, zeros_n)
    return _stage_e(acc2, W_out, b_out2)
